# baseline (speedup 1.0000x reference)
import functools

import jax
import jax.numpy as jnp
from jax import lax
from jax.experimental import pallas as pl
from jax.experimental.pallas import tpu as pltpu
from jax.experimental.pallas import tpu_sc as plsc

_DIM = 64
_BATCH = 16384
_HIST = 200
_B = _BATCH * _HIST
_NW = 32
_CB = 128
_NCH = _B // _CB
_CPW = _NCH // _NW
_BT = _BATCH // _CB

_mesh = plsc.VectorSubcoreMesh(core_axis_name="c", subcore_axis_name="s")


@functools.partial(
    pl.kernel,
    mesh=_mesh,
    out_type=jax.ShapeDtypeStruct((_HIST, 8, _BT, 8, _CB), jnp.float32),
    scratch_types=[
        pltpu.VMEM((2, _CB), jnp.int32),
        pltpu.VMEM((2, _CB, _DIM), jnp.float32),
        pltpu.VMEM((2, 8, 8, _CB), jnp.float32),
        pltpu.SemaphoreType.DMA,
        pltpu.SemaphoreType.DMA,
        pltpu.SemaphoreType.DMA,
    ],
    compiler_params=pltpu.CompilerParams(use_tc_tiling_on_sc=False,
                                         needs_layout_passes=False),
)
def _emb(x_hbm, tab_hbm, out_hbm, idx_v, rows_v, stage_v, isem, gsem, osem):
    wid = lax.axis_index("s") * 2 + lax.axis_index("c")
    g0 = wid * _CPW

    biota = lax.iota(jnp.int32, 16)

    def idx_load(g, slot, sem):
        return pltpu.async_copy(x_hbm.at[pl.ds(g * _CB, _CB)],
                                idx_v.at[slot], sem)

    def gather_start(slot):
        return pltpu.async_copy(tab_hbm.at[idx_v.at[slot]], rows_v.at[slot],
                                gsem)

    def store_start(g, slot):
        s = g // _BT
        bt = g % _BT
        return pltpu.async_copy(stage_v.at[slot],
                                out_hbm.at[s, :, bt], osem)

    def wait_rows(sem, slot):
        pltpu.make_async_copy(tab_hbm.at[pl.ds(0, _CB)], rows_v.at[slot],
                              sem).wait()

    def wait_idx(slot):
        pltpu.make_async_copy(x_hbm.at[pl.ds(0, _CB)], idx_v.at[slot],
                              isem).wait()

    def transpose(slot):
        rows2 = rows_v.at[slot]

        def tl(lh, carry):
            for ll in range(8):
                lvec = jnp.full((16,), ll, jnp.int32) + lh * 8
                for j0 in range(_CB // 16):
                    stage_v[slot, lh, ll, pl.ds(j0 * 16, 16)] = (
                        plsc.load_gather(rows2, [biota + j0 * 16, lvec]))
            return carry

        lax.fori_loop(0, 8, tl, 0)

    idx_load(g0, 0, isem)
    wait_idx(0)
    gather_start(0)
    idx_load(g0 + 1, 1, isem)

    wait_rows(gsem, 0)
    wait_idx(1)
    gather_start(1)
    idx_load(g0 + 2, 0, isem)
    transpose(0)
    store_start(g0, 0)

    wait_rows(gsem, 1)
    wait_idx(0)
    gather_start(0)
    idx_load(g0 + 3, 1, isem)
    transpose(1)
    store_start(g0 + 1, 1)

    def body(i, carry):
        g = g0 + i
        slot = i % 2
        wait_rows(osem, slot)
        wait_rows(gsem, slot)
        wait_idx(1 - slot)
        gather_start(1 - slot)
        idx_load(g + 2, slot, isem)
        transpose(slot)
        store_start(g, slot)
        return carry

    lax.fori_loop(2, _CPW - 2, body, 0)

    i = _CPW - 2
    slot = i % 2
    wait_rows(osem, slot)
    wait_rows(gsem, slot)
    wait_idx(1 - slot)
    gather_start(1 - slot)
    transpose(slot)
    store_start(g0 + i, slot)

    i = _CPW - 1
    slot = i % 2
    wait_rows(osem, slot)
    wait_rows(gsem, slot)
    transpose(slot)
    store_start(g0 + i, slot)

    wait_rows(osem, 0)
    wait_rows(osem, 1)


def kernel(x, table):
    flat = x.T.reshape(_B)
    out5 = _emb(flat, table)
    return out5.transpose(2, 4, 0, 1, 3).reshape(_BATCH, _HIST, _DIM)


# speedup vs baseline: 1.7440x; 1.7440x over previous
import functools

import jax
import jax.numpy as jnp
from jax import lax
from jax.experimental import pallas as pl
from jax.experimental.pallas import tpu as pltpu
from jax.experimental.pallas import tpu_sc as plsc

_DIM = 64
_BATCH = 16384
_HIST = 200
_B = _BATCH * _HIST
_NW = 32
_CB = 128
_NCH = _B // _CB
_CPW = _NCH // _NW
_BT = _BATCH // _CB

_mesh = plsc.VectorSubcoreMesh(core_axis_name="c", subcore_axis_name="s")


@functools.partial(
    pl.kernel,
    mesh=_mesh,
    out_type=jax.ShapeDtypeStruct((_HIST, 8, _BT, 8, _CB), jnp.float32),
    scratch_types=[
        pltpu.VMEM((2, _CB), jnp.int32),
        pltpu.VMEM((2, _CB, _DIM), jnp.float32),
        pltpu.VMEM((2, 8, 8, _CB), jnp.float32),
        pltpu.SemaphoreType.DMA,
        pltpu.SemaphoreType.DMA,
        pltpu.SemaphoreType.DMA,
    ],
    compiler_params=pltpu.CompilerParams(use_tc_tiling_on_sc=False,
                                         needs_layout_passes=False),
)
def _emb(x_hbm, tab_hbm, out_hbm, idx_v, rows_v, stage_v, isem, gsem, osem):
    wid = lax.axis_index("s") * 2 + lax.axis_index("c")
    g0 = wid * _CPW

    biota = lax.iota(jnp.int32, 16)

    def idx_load(g, slot, sem):
        return pltpu.async_copy(x_hbm.at[pl.ds(g * _CB, _CB)],
                                idx_v.at[slot], sem)

    def gather_start(slot):
        return pltpu.async_copy(tab_hbm.at[idx_v.at[slot]], rows_v.at[slot],
                                gsem)

    def store_start(g, slot):
        s = g // _BT
        bt = g % _BT
        return pltpu.async_copy(stage_v.at[slot],
                                out_hbm.at[s, :, bt], osem)

    def wait_rows(sem, slot):
        pltpu.make_async_copy(tab_hbm.at[pl.ds(0, _CB)], rows_v.at[slot],
                              sem).wait()

    def wait_idx(slot):
        pltpu.make_async_copy(x_hbm.at[pl.ds(0, _CB)], idx_v.at[slot],
                              isem).wait()

    def transpose(slot):
        rows2 = rows_v.at[slot]

        @plsc.parallel_loop(0, _DIM, unroll=2)
        def _t(l):
            lh = l // 8
            ll = l % 8
            lvec = jnp.full((16,), 0, jnp.int32) + l
            for j0 in range(_CB // 16):
                stage_v[slot, lh, ll, pl.ds(j0 * 16, 16)] = (
                    plsc.load_gather(rows2, [biota + j0 * 16, lvec]))

    idx_load(g0, 0, isem)
    wait_idx(0)
    gather_start(0)
    idx_load(g0 + 1, 1, isem)

    wait_rows(gsem, 0)
    wait_idx(1)
    gather_start(1)
    idx_load(g0 + 2, 0, isem)
    transpose(0)
    store_start(g0, 0)

    wait_rows(gsem, 1)
    wait_idx(0)
    gather_start(0)
    idx_load(g0 + 3, 1, isem)
    transpose(1)
    store_start(g0 + 1, 1)

    def body(i, carry):
        g = g0 + i
        slot = i % 2
        wait_rows(osem, slot)
        wait_rows(gsem, slot)
        wait_idx(1 - slot)
        gather_start(1 - slot)
        idx_load(g + 2, slot, isem)
        transpose(slot)
        store_start(g, slot)
        return carry

    lax.fori_loop(2, _CPW - 2, body, 0)

    i = _CPW - 2
    slot = i % 2
    wait_rows(osem, slot)
    wait_rows(gsem, slot)
    wait_idx(1 - slot)
    gather_start(1 - slot)
    transpose(slot)
    store_start(g0 + i, slot)

    i = _CPW - 1
    slot = i % 2
    wait_rows(osem, slot)
    wait_rows(gsem, slot)
    transpose(slot)
    store_start(g0 + i, slot)

    wait_rows(osem, 0)
    wait_rows(osem, 1)


def kernel(x, table):
    flat = x.T.reshape(_B)
    out5 = _emb(flat, table)
    return out5.transpose(2, 4, 0, 1, 3).reshape(_BATCH, _HIST, _DIM)


# speedup vs baseline: 3.6070x; 2.0683x over previous
import functools

import jax
import jax.numpy as jnp
from jax import lax
from jax.experimental import pallas as pl
from jax.experimental.pallas import tpu as pltpu
from jax.experimental.pallas import tpu_sc as plsc

_DIM = 64
_BATCH = 16384
_HIST = 200
_B = _BATCH * _HIST
_NW = 32
_CB = 128
_NCH = _B // _CB
_CPW = _NCH // _NW
_BT = _BATCH // _CB

_mesh = plsc.VectorSubcoreMesh(core_axis_name="c", subcore_axis_name="s")


@functools.partial(
    pl.kernel,
    mesh=_mesh,
    out_type=jax.ShapeDtypeStruct((_HIST, 8, _BT, 8, _CB), jnp.float32),
    scratch_types=[
        pltpu.VMEM((2, _CB), jnp.int32),
        pltpu.VMEM((2, _CB, _DIM), jnp.float32),
        pltpu.VMEM((2, 8, 8, _CB + 1), jnp.float32),
        pltpu.SemaphoreType.DMA,
        pltpu.SemaphoreType.DMA,
        pltpu.SemaphoreType.DMA,
    ],
    compiler_params=pltpu.CompilerParams(use_tc_tiling_on_sc=False,
                                         needs_layout_passes=False),
)
def _emb(x_hbm, tab_hbm, out_hbm, idx_v, rows_v, stage_v, isem, gsem, osem):
    wid = lax.axis_index("s") * 2 + lax.axis_index("c")
    g0 = wid * _CPW

    biota = lax.iota(jnp.int32, 16)

    def idx_load(g, slot, sem):
        return pltpu.async_copy(x_hbm.at[pl.ds(g * _CB, _CB)],
                                idx_v.at[slot], sem)

    def gather_start(slot):
        return pltpu.async_copy(tab_hbm.at[idx_v.at[slot]], rows_v.at[slot],
                                gsem)

    def store_start(g, slot):
        s = g // _BT
        bt = g % _BT
        return pltpu.async_copy(stage_v.at[slot, :, :, pl.ds(0, _CB)],
                                out_hbm.at[s, :, bt], osem)

    def wait_rows(sem, slot):
        pltpu.make_async_copy(tab_hbm.at[pl.ds(0, _CB)], rows_v.at[slot],
                              sem).wait()

    def wait_idx(slot):
        pltpu.make_async_copy(x_hbm.at[pl.ds(0, _CB)], idx_v.at[slot],
                              isem).wait()

    lhv = [(biota + k * 16) // 8 for k in range(_DIM // 16)]
    llv = [(biota + k * 16) % 8 for k in range(_DIM // 16)]

    def transpose(slot):
        rows2 = rows_v.at[slot]
        stage3 = stage_v.at[slot]

        @plsc.parallel_loop(0, _CB, unroll=4)
        def _t(b):
            bfull = jnp.full((16,), 0, jnp.int32) + b
            for k in range(_DIM // 16):
                vals = rows2[b, pl.ds(k * 16, 16)]
                plsc.store_scatter(stage3, [lhv[k], llv[k], bfull], vals)

    idx_load(g0, 0, isem)
    wait_idx(0)
    gather_start(0)
    idx_load(g0 + 1, 1, isem)

    wait_rows(gsem, 0)
    wait_idx(1)
    gather_start(1)
    idx_load(g0 + 2, 0, isem)
    transpose(0)
    store_start(g0, 0)

    wait_rows(gsem, 1)
    wait_idx(0)
    gather_start(0)
    idx_load(g0 + 3, 1, isem)
    transpose(1)
    store_start(g0 + 1, 1)

    def body(i, carry):
        g = g0 + i
        slot = i % 2
        wait_rows(osem, slot)
        wait_rows(gsem, slot)
        wait_idx(1 - slot)
        gather_start(1 - slot)
        idx_load(g + 2, slot, isem)
        transpose(slot)
        store_start(g, slot)
        return carry

    lax.fori_loop(2, _CPW - 2, body, 0)

    i = _CPW - 2
    slot = i % 2
    wait_rows(osem, slot)
    wait_rows(gsem, slot)
    wait_idx(1 - slot)
    gather_start(1 - slot)
    transpose(slot)
    store_start(g0 + i, slot)

    i = _CPW - 1
    slot = i % 2
    wait_rows(osem, slot)
    wait_rows(gsem, slot)
    transpose(slot)
    store_start(g0 + i, slot)

    wait_rows(osem, 0)
    wait_rows(osem, 1)


def kernel(x, table):
    flat = x.T.reshape(_B)
    out5 = _emb(flat, table)
    return out5.transpose(2, 4, 0, 1, 3).reshape(_BATCH, _HIST, _DIM)


# speedup vs baseline: 4.3785x; 1.2139x over previous
import functools

import jax
import jax.numpy as jnp
from jax import lax
from jax.experimental import pallas as pl
from jax.experimental.pallas import tpu as pltpu
from jax.experimental.pallas import tpu_sc as plsc

_DIM = 64
_BATCH = 16384
_HIST = 200
_B = _BATCH * _HIST
_NW = 32
_NBT = 2
_CB = 128 * _NBT
_NCH = _B // _CB
_CPW = _NCH // _NW
_BT = _BATCH // 128
_TPH = _BT // _NBT

_mesh = plsc.VectorSubcoreMesh(core_axis_name="c", subcore_axis_name="s")


@functools.partial(
    pl.kernel,
    mesh=_mesh,
    out_type=jax.ShapeDtypeStruct((_HIST, 8, _BT, 8, 128), jnp.float32),
    scratch_types=[
        pltpu.VMEM((2, _CB), jnp.int32),
        pltpu.VMEM((2, _CB, _DIM), jnp.float32),
        pltpu.VMEM((2, 8, _NBT, 8, 129), jnp.float32),
        pltpu.SemaphoreType.DMA,
        pltpu.SemaphoreType.DMA,
        pltpu.SemaphoreType.DMA,
    ],
    compiler_params=pltpu.CompilerParams(use_tc_tiling_on_sc=False,
                                         needs_layout_passes=False),
)
def _emb(x_hbm, tab_hbm, out_hbm, idx_v, rows_v, stage_v, isem, gsem, osem):
    wid = lax.axis_index("s") * 2 + lax.axis_index("c")
    g0 = wid * _CPW

    biota = lax.iota(jnp.int32, 16)

    def idx_load(g, slot, sem):
        return pltpu.async_copy(x_hbm.at[pl.ds(g * _CB, _CB)],
                                idx_v.at[slot], sem)

    def gather_start(slot):
        return pltpu.async_copy(tab_hbm.at[idx_v.at[slot]], rows_v.at[slot],
                                gsem)

    def store_start(g, slot):
        s = g // _TPH
        bt0 = (g % _TPH) * _NBT
        return pltpu.async_copy(stage_v.at[slot, :, :, :, pl.ds(0, 128)],
                                out_hbm.at[s, :, pl.ds(bt0, _NBT)], osem)

    def wait_rows(sem, slot):
        pltpu.make_async_copy(tab_hbm.at[pl.ds(0, _CB)], rows_v.at[slot],
                              sem).wait()

    def wait_idx(slot):
        pltpu.make_async_copy(x_hbm.at[pl.ds(0, _CB)], idx_v.at[slot],
                              isem).wait()

    lhv = [(biota + k * 16) // 8 for k in range(_DIM // 16)]
    llv = [(biota + k * 16) % 8 for k in range(_DIM // 16)]

    def transpose(slot):
        rows2 = rows_v.at[slot]
        stage4 = stage_v.at[slot]

        @plsc.parallel_loop(0, _CB, unroll=4)
        def _t(b):
            btfull = jnp.full((16,), 0, jnp.int32) + (b // 128)
            bfull = jnp.full((16,), 0, jnp.int32) + (b % 128)
            for k in range(_DIM // 16):
                vals = rows2[b, pl.ds(k * 16, 16)]
                plsc.store_scatter(stage4, [lhv[k], btfull, llv[k], bfull],
                                   vals)

    idx_load(g0, 0, isem)
    wait_idx(0)
    gather_start(0)
    idx_load(g0 + 1, 1, isem)

    wait_rows(gsem, 0)
    wait_idx(1)
    gather_start(1)
    idx_load(g0 + 2, 0, isem)
    transpose(0)
    store_start(g0, 0)

    wait_rows(gsem, 1)
    wait_idx(0)
    gather_start(0)
    idx_load(g0 + 3, 1, isem)
    transpose(1)
    store_start(g0 + 1, 1)

    def body(i, carry):
        g = g0 + i
        slot = i % 2
        wait_rows(osem, slot)
        wait_rows(gsem, slot)
        wait_idx(1 - slot)
        gather_start(1 - slot)
        idx_load(g + 2, slot, isem)
        transpose(slot)
        store_start(g, slot)
        return carry

    lax.fori_loop(2, _CPW - 2, body, 0)

    i = _CPW - 2
    slot = i % 2
    wait_rows(osem, slot)
    wait_rows(gsem, slot)
    wait_idx(1 - slot)
    gather_start(1 - slot)
    transpose(slot)
    store_start(g0 + i, slot)

    i = _CPW - 1
    slot = i % 2
    wait_rows(osem, slot)
    wait_rows(gsem, slot)
    transpose(slot)
    store_start(g0 + i, slot)

    wait_rows(osem, 0)
    wait_rows(osem, 1)


def kernel(x, table):
    flat = x.T.reshape(_B)
    out5 = _emb(flat, table)
    return out5.transpose(2, 4, 0, 1, 3).reshape(_BATCH, _HIST, _DIM)


# speedup vs baseline: 4.6388x; 1.0594x over previous
import functools

import jax
import jax.numpy as jnp
from jax import lax
from jax.experimental import pallas as pl
from jax.experimental.pallas import tpu as pltpu
from jax.experimental.pallas import tpu_sc as plsc

_DIM = 64
_BATCH = 16384
_HIST = 200
_B = _BATCH * _HIST
_NW = 32
_NBT = 2
_CB = 128 * _NBT
_NCH = _B // _CB
_CPW = _NCH // _NW
_BT = _BATCH // 128
_TPH = _BT // _NBT

_mesh = plsc.VectorSubcoreMesh(core_axis_name="c", subcore_axis_name="s")


@functools.partial(
    pl.kernel,
    mesh=_mesh,
    out_type=jax.ShapeDtypeStruct((_HIST, 8, _BT, 8, 128), jnp.float32),
    scratch_types=[
        pltpu.VMEM((3, _CB), jnp.int32),
        pltpu.VMEM((3, _CB, _DIM), jnp.float32),
        pltpu.VMEM((2, 8, _NBT, 8, 129), jnp.float32),
        pltpu.SemaphoreType.DMA,
        pltpu.SemaphoreType.DMA,
        pltpu.SemaphoreType.DMA,
    ],
    compiler_params=pltpu.CompilerParams(use_tc_tiling_on_sc=False,
                                         needs_layout_passes=False),
)
def _emb(x_hbm, tab_hbm, out_hbm, idx_v, rows_v, stage_v, isem, gsem, osem):
    wid = lax.axis_index("s") * 2 + lax.axis_index("c")
    g0 = wid * _CPW

    biota = lax.iota(jnp.int32, 16)

    def idx_load(g, slot, sem):
        return pltpu.async_copy(x_hbm.at[pl.ds(g * _CB, _CB)],
                                idx_v.at[slot], sem)

    def gather_start(slot):
        return pltpu.async_copy(tab_hbm.at[idx_v.at[slot]], rows_v.at[slot],
                                gsem)

    def store_start(g, slot):
        s = g // _TPH
        bt0 = (g % _TPH) * _NBT
        return pltpu.async_copy(stage_v.at[slot, :, :, :, pl.ds(0, 128)],
                                out_hbm.at[s, :, pl.ds(bt0, _NBT)], osem)

    def wait_rows(sem, slot):
        pltpu.make_async_copy(tab_hbm.at[pl.ds(0, _CB)], rows_v.at[slot],
                              sem).wait()

    def wait_idx(slot):
        pltpu.make_async_copy(x_hbm.at[pl.ds(0, _CB)], idx_v.at[slot],
                              isem).wait()

    lhv = [(biota + k * 16) // 8 for k in range(_DIM // 16)]
    llv = [(biota + k * 16) % 8 for k in range(_DIM // 16)]

    def transpose_src(slot, sslot):
        rows2 = rows_v.at[slot]
        stage4 = stage_v.at[sslot]

        @plsc.parallel_loop(0, _CB, unroll=4)
        def _t(b):
            btfull = jnp.full((16,), 0, jnp.int32) + (b // 128)
            bfull = jnp.full((16,), 0, jnp.int32) + (b % 128)
            for k in range(_DIM // 16):
                vals = rows2[b, pl.ds(k * 16, 16)]
                plsc.store_scatter(stage4, [lhv[k], btfull, llv[k], bfull],
                                   vals)

    idx_load(g0, 0, isem)
    wait_idx(0)
    gather_start(0)
    idx_load(g0 + 1, 1, isem)
    wait_idx(1)
    gather_start(1)
    idx_load(g0 + 2, 2, isem)

    for i in range(2):
        slot = i % 3
        wait_rows(gsem, slot)
        wait_idx((i + 2) % 3)
        gather_start((i + 2) % 3)
        idx_load(g0 + i + 3, slot, isem)
        transpose_src(slot, i % 2)
        store_start(g0 + i, i % 2)

    def body(i, carry):
        g = g0 + i
        slot = i % 3
        sslot = i % 2
        wait_rows(osem, sslot)
        wait_rows(gsem, slot)
        wait_idx((i + 2) % 3)
        gather_start((i + 2) % 3)
        idx_load(g + 3, slot, isem)
        transpose_src(slot, sslot)
        store_start(g, sslot)
        return carry

    lax.fori_loop(2, _CPW - 3, body, 0)

    i = _CPW - 3
    wait_rows(osem, i % 2)
    wait_rows(gsem, i % 3)
    wait_idx((i + 2) % 3)
    gather_start((i + 2) % 3)
    transpose_src(i % 3, i % 2)
    store_start(g0 + i, i % 2)

    for i in range(_CPW - 2, _CPW):
        wait_rows(osem, i % 2)
        wait_rows(gsem, i % 3)
        transpose_src(i % 3, i % 2)
        store_start(g0 + i, i % 2)

    wait_rows(osem, 0)
    wait_rows(osem, 1)


def kernel(x, table):
    flat = x.T.reshape(_B)
    out5 = _emb(flat, table)
    return out5.transpose(2, 4, 0, 1, 3).reshape(_BATCH, _HIST, _DIM)


# speedup vs baseline: 4.6402x; 1.0003x over previous
import functools

import jax
import jax.numpy as jnp
from jax import lax
from jax.experimental import pallas as pl
from jax.experimental.pallas import tpu as pltpu
from jax.experimental.pallas import tpu_sc as plsc

_DIM = 64
_BATCH = 16384
_HIST = 200
_B = _BATCH * _HIST
_NW = 32
_NBT = 2
_CB = 128 * _NBT
_NCH = _B // _CB
_CPW = _NCH // _NW
_BT = _BATCH // 128
_TPH = _BT // _NBT

_mesh = plsc.VectorSubcoreMesh(core_axis_name="c", subcore_axis_name="s")


@functools.partial(
    pl.kernel,
    mesh=_mesh,
    out_type=jax.ShapeDtypeStruct((_HIST, 8, _BT, 8, 128), jnp.float32),
    scratch_types=[
        pltpu.VMEM((3, _CB), jnp.int32),
        pltpu.VMEM((3, _CB, _DIM), jnp.float32),
        pltpu.VMEM((2, 8, _NBT, 8, 129), jnp.float32),
        pltpu.SemaphoreType.DMA,
        pltpu.SemaphoreType.DMA,
        pltpu.SemaphoreType.DMA,
    ],
    compiler_params=pltpu.CompilerParams(use_tc_tiling_on_sc=False,
                                         needs_layout_passes=False),
)
def _emb(x_hbm, tab_hbm, out_hbm, idx_v, rows_v, stage_v, isem, gsem, osem):
    wid = lax.axis_index("s") * 2 + lax.axis_index("c")
    g0 = wid * _CPW

    biota = lax.iota(jnp.int32, 16)

    def idx_load(g, slot, sem):
        return pltpu.async_copy(x_hbm.at[pl.ds(g * _CB, _CB)],
                                idx_v.at[slot], sem)

    def gather_start(slot):
        return pltpu.async_copy(tab_hbm.at[idx_v.at[slot]], rows_v.at[slot],
                                gsem)

    def store_start(g, slot):
        s = g // _TPH
        bt0 = (g % _TPH) * _NBT
        return pltpu.async_copy(stage_v.at[slot, :, :, :, pl.ds(0, 128)],
                                out_hbm.at[s, :, pl.ds(bt0, _NBT)], osem)

    def wait_rows(sem, slot):
        pltpu.make_async_copy(tab_hbm.at[pl.ds(0, _CB)], rows_v.at[slot],
                              sem).wait()

    def wait_idx(slot):
        pltpu.make_async_copy(x_hbm.at[pl.ds(0, _CB)], idx_v.at[slot],
                              isem).wait()

    lhv = [(biota + k * 16) // 8 for k in range(_DIM // 16)]
    llv = [(biota + k * 16) % 8 for k in range(_DIM // 16)]

    def transpose_src(slot, sslot):
        rows2 = rows_v.at[slot]
        stage4 = stage_v.at[sslot]

        @plsc.parallel_loop(0, _CB, unroll=4)
        def _t(b):
            btfull = jnp.full((16,), 0, jnp.int32) + (b // 128)
            bfull = jnp.full((16,), 0, jnp.int32) + (b % 128)
            for k in range(_DIM // 16):
                vals = rows2[b, pl.ds(k * 16, 16)]
                plsc.store_scatter(stage4, [lhv[k], btfull, llv[k], bfull],
                                   vals)

    idx_load(g0, 0, isem)
    wait_idx(0)
    gather_start(0)
    idx_load(g0 + 1, 1, isem)
    wait_idx(1)
    gather_start(1)
    idx_load(g0 + 2, 2, isem)

    for i in range(2):
        slot = i % 3
        wait_rows(gsem, slot)
        wait_idx((i + 2) % 3)
        gather_start((i + 2) % 3)
        idx_load(g0 + i + 3, slot, isem)
        transpose_src(slot, i % 2)
        store_start(g0 + i, i % 2)

    def body(i, carry):
        g = g0 + i
        slot = i % 3
        sslot = i % 2
        wait_rows(gsem, slot)
        wait_idx((i + 2) % 3)
        gather_start((i + 2) % 3)
        idx_load(g + 3, slot, isem)
        wait_rows(osem, sslot)
        transpose_src(slot, sslot)
        store_start(g, sslot)
        return carry

    lax.fori_loop(2, _CPW - 3, body, 0)

    i = _CPW - 3
    wait_rows(osem, i % 2)
    wait_rows(gsem, i % 3)
    wait_idx((i + 2) % 3)
    gather_start((i + 2) % 3)
    transpose_src(i % 3, i % 2)
    store_start(g0 + i, i % 2)

    for i in range(_CPW - 2, _CPW):
        wait_rows(osem, i % 2)
        wait_rows(gsem, i % 3)
        transpose_src(i % 3, i % 2)
        store_start(g0 + i, i % 2)

    wait_rows(osem, 0)
    wait_rows(osem, 1)


def kernel(x, table):
    flat = x.T.reshape(_B)
    out5 = _emb(flat, table)
    return out5.transpose(2, 4, 0, 1, 3).reshape(_BATCH, _HIST, _DIM)


# speedup vs baseline: 4.6492x; 1.0020x over previous
import functools

import jax
import jax.numpy as jnp
from jax import lax
from jax.experimental import pallas as pl
from jax.experimental.pallas import tpu as pltpu
from jax.experimental.pallas import tpu_sc as plsc

_DIM = 64
_BATCH = 16384
_HIST = 200
_B = _BATCH * _HIST
_NW = 32
_NBT = 2
_CB = 128 * _NBT
_NCH = _B // _CB
_CPW = _NCH // _NW
_BT = _BATCH // 128
_TPH = _BT // _NBT

_mesh = plsc.VectorSubcoreMesh(core_axis_name="c", subcore_axis_name="s")


@functools.partial(
    pl.kernel,
    mesh=_mesh,
    out_type=jax.ShapeDtypeStruct((_HIST, 8, _BT, 8, 128), jnp.float32),
    scratch_types=[
        pltpu.VMEM((4, _CB), jnp.int32),
        pltpu.VMEM((4, _CB, _DIM), jnp.float32),
        pltpu.VMEM((2, 8, _NBT, 8, 129), jnp.float32),
        pltpu.SemaphoreType.DMA,
        pltpu.SemaphoreType.DMA,
        pltpu.SemaphoreType.DMA,
    ],
    compiler_params=pltpu.CompilerParams(use_tc_tiling_on_sc=False,
                                         needs_layout_passes=False),
)
def _emb(x_hbm, tab_hbm, out_hbm, idx_v, rows_v, stage_v, isem, gsem, osem):
    wid = lax.axis_index("s") * 2 + lax.axis_index("c")
    g0 = wid * _CPW

    biota = lax.iota(jnp.int32, 16)

    def idx_load(g, slot, sem):
        return pltpu.async_copy(x_hbm.at[pl.ds(g * _CB, _CB)],
                                idx_v.at[slot], sem)

    def gather_start(slot):
        return pltpu.async_copy(tab_hbm.at[idx_v.at[slot]], rows_v.at[slot],
                                gsem)

    def store_start(g, slot):
        s = g // _TPH
        bt0 = (g % _TPH) * _NBT
        return pltpu.async_copy(stage_v.at[slot, :, :, :, pl.ds(0, 128)],
                                out_hbm.at[s, :, pl.ds(bt0, _NBT)], osem)

    def wait_rows(sem, slot):
        pltpu.make_async_copy(tab_hbm.at[pl.ds(0, _CB)], rows_v.at[slot],
                              sem).wait()

    def wait_idx(slot):
        pltpu.make_async_copy(x_hbm.at[pl.ds(0, _CB)], idx_v.at[slot],
                              isem).wait()

    lhv = [(biota + k * 16) // 8 for k in range(_DIM // 16)]
    llv = [(biota + k * 16) % 8 for k in range(_DIM // 16)]

    def transpose_src(slot, sslot):
        rows2 = rows_v.at[slot]
        stage4 = stage_v.at[sslot]

        @plsc.parallel_loop(0, _CB, unroll=4)
        def _t(b):
            btfull = jnp.full((16,), 0, jnp.int32) + (b // 128)
            bfull = jnp.full((16,), 0, jnp.int32) + (b % 128)
            for k in range(_DIM // 16):
                vals = rows2[b, pl.ds(k * 16, 16)]
                plsc.store_scatter(stage4, [lhv[k], btfull, llv[k], bfull],
                                   vals)

    for k in range(3):
        idx_load(g0 + k, k, isem)
        wait_idx(k)
        gather_start(k)
    idx_load(g0 + 3, 3, isem)

    for i in range(2):
        slot = i % 4
        wait_rows(gsem, slot)
        wait_idx((i + 3) % 4)
        gather_start((i + 3) % 4)
        idx_load(g0 + i + 4, slot, isem)
        transpose_src(slot, i % 2)
        store_start(g0 + i, i % 2)

    def body(i, carry):
        g = g0 + i
        slot = i % 4
        sslot = i % 2
        wait_rows(gsem, slot)
        wait_idx((i + 3) % 4)
        gather_start((i + 3) % 4)
        idx_load(g + 4, slot, isem)
        wait_rows(osem, sslot)
        transpose_src(slot, sslot)
        store_start(g, sslot)
        return carry

    lax.fori_loop(2, _CPW - 4, body, 0)

    i = _CPW - 4
    wait_rows(gsem, i % 4)
    wait_idx((i + 3) % 4)
    gather_start((i + 3) % 4)
    wait_rows(osem, i % 2)
    transpose_src(i % 4, i % 2)
    store_start(g0 + i, i % 2)

    for i in range(_CPW - 3, _CPW):
        wait_rows(gsem, i % 4)
        wait_rows(osem, i % 2)
        transpose_src(i % 4, i % 2)
        store_start(g0 + i, i % 2)

    wait_rows(osem, 0)
    wait_rows(osem, 1)


def kernel(x, table):
    flat = x.T.reshape(_B)
    out5 = _emb(flat, table)
    return out5.transpose(2, 4, 0, 1, 3).reshape(_BATCH, _HIST, _DIM)


# speedup vs baseline: 4.6532x; 1.0008x over previous
import functools

import jax
import jax.numpy as jnp
from jax import lax
from jax.experimental import pallas as pl
from jax.experimental.pallas import tpu as pltpu
from jax.experimental.pallas import tpu_sc as plsc

_DIM = 64
_BATCH = 16384
_HIST = 200
_B = _BATCH * _HIST
_NW = 32
_NBT = 2
_CB = 128 * _NBT
_NCH = _B // _CB
_CPW = _NCH // _NW
_BT = _BATCH // 128
_TPH = _BT // _NBT

_mesh = plsc.VectorSubcoreMesh(core_axis_name="c", subcore_axis_name="s")


@functools.partial(
    pl.kernel,
    mesh=_mesh,
    out_type=jax.ShapeDtypeStruct((_HIST, 8, _BT, 8, 128), jnp.float32),
    scratch_types=[
        pltpu.VMEM((3, _CB), jnp.int32),
        pltpu.VMEM((3, _CB, _DIM), jnp.float32),
        pltpu.VMEM((2, 8, _NBT, 8, 129), jnp.float32),
        pltpu.SemaphoreType.DMA,
        pltpu.SemaphoreType.DMA,
        pltpu.SemaphoreType.DMA,
    ],
    compiler_params=pltpu.CompilerParams(use_tc_tiling_on_sc=False,
                                         needs_layout_passes=False),
)
def _emb(x_hbm, tab_hbm, out_hbm, idx_v, rows_v, stage_v, isem, gsem, osem):
    wid = lax.axis_index("s") * 2 + lax.axis_index("c")
    g0 = wid * _CPW

    biota = lax.iota(jnp.int32, 16)

    def idx_load(g, slot, sem):
        return pltpu.async_copy(x_hbm.at[pl.ds(g * _CB, _CB)],
                                idx_v.at[slot], sem)

    def gather_start(slot):
        return pltpu.async_copy(tab_hbm.at[idx_v.at[slot]], rows_v.at[slot],
                                gsem)

    def store_start(g, slot):
        s = g // _TPH
        bt0 = (g % _TPH) * _NBT
        return pltpu.async_copy(stage_v.at[slot, :, :, :, pl.ds(0, 128)],
                                out_hbm.at[s, :, pl.ds(bt0, _NBT)], osem)

    def wait_rows(sem, slot):
        pltpu.make_async_copy(tab_hbm.at[pl.ds(0, _CB)], rows_v.at[slot],
                              sem).wait()

    def wait_idx(slot):
        pltpu.make_async_copy(x_hbm.at[pl.ds(0, _CB)], idx_v.at[slot],
                              isem).wait()

    lhv = [(biota + k * 16) // 8 for k in range(_DIM // 16)]
    llv = [(biota + k * 16) % 8 for k in range(_DIM // 16)]

    def transpose_src(slot, sslot):
        rows2 = rows_v.at[slot]
        stage4 = stage_v.at[sslot]

        @plsc.parallel_loop(0, _CB, unroll=4)
        def _t(b):
            btfull = jnp.full((16,), 0, jnp.int32) + (b // 128)
            bfull = jnp.full((16,), 0, jnp.int32) + (b % 128)
            for k in range(_DIM // 16):
                vals = rows2[b, pl.ds(k * 16, 16)]
                plsc.store_scatter(stage4, [lhv[k], btfull, llv[k], bfull],
                                   vals)

    idx_load(g0, 0, isem)
    wait_idx(0)
    gather_start(0)
    idx_load(g0 + 1, 1, isem)
    wait_idx(1)
    gather_start(1)
    idx_load(g0 + 2, 2, isem)

    for i in range(2):
        slot = i % 3
        wait_rows(gsem, slot)
        wait_idx((i + 2) % 3)
        gather_start((i + 2) % 3)
        idx_load(g0 + i + 3, slot, isem)
        transpose_src(slot, i % 2)
        store_start(g0 + i, i % 2)

    def body(i, carry):
        g = g0 + i
        slot = i % 3
        sslot = i % 2
        wait_rows(gsem, slot)
        wait_idx((i + 2) % 3)
        gather_start((i + 2) % 3)
        idx_load(g + 3, slot, isem)
        wait_rows(osem, sslot)
        transpose_src(slot, sslot)
        store_start(g, sslot)
        return carry

    lax.fori_loop(2, _CPW - 3, body, 0)

    i = _CPW - 3
    wait_rows(gsem, i % 3)
    wait_idx((i + 2) % 3)
    gather_start((i + 2) % 3)
    wait_rows(osem, i % 2)
    transpose_src(i % 3, i % 2)
    store_start(g0 + i, i % 2)

    for i in range(_CPW - 2, _CPW):
        wait_rows(gsem, i % 3)
        wait_rows(osem, i % 2)
        transpose_src(i % 3, i % 2)
        store_start(g0 + i, i % 2)

    wait_rows(osem, 0)
    wait_rows(osem, 1)


def kernel(x, table):
    flat = x.T.reshape(_B)
    out5 = _emb(flat, table)
    return out5.transpose(2, 4, 0, 1, 3).reshape(_BATCH, _HIST, _DIM)
